# baseline (speedup 1.0000x reference)
import functools

import jax
import jax.numpy as jnp
from jax import lax
from jax.experimental import pallas as pl
from jax.experimental.pallas import tpu as pltpu
from jax.experimental.pallas import tpu_sc as plsc

R = 128
N = 8192
L = 16
NC = 2
NS = 16
NW = NC * NS
RPW = R // NW
HALVES = 2
SEG = RPW * HALVES
N2 = N // HALVES
U = 8
T2 = N2 // (L * U)

_mesh = plsc.VectorSubcoreMesh(core_axis_name="c", subcore_axis_name="s")


@functools.partial(
    pl.kernel,
    out_type=jax.ShapeDtypeStruct((R,), jnp.int32),
    mesh=_mesh,
    compiler_params=pltpu.CompilerParams(
        needs_layout_passes=False,
        disable_bounds_checks=True,
        disable_semaphore_checks=True,
    ),
    scratch_types=[
        pltpu.VMEM((SEG, N2), jnp.float32),
        pltpu.VMEM((L,), jnp.int32),
    ]
    + [pltpu.SemaphoreType.DMA] * (SEG + 1),
)
def _argmin_sc(x_hbm, out_hbm, buf, res_v, *sems):
    cid = lax.axis_index("c")
    sid = lax.axis_index("s")
    base = (cid * NS + sid) * RPW
    iota = lax.iota(jnp.int32, L)
    resvec = jnp.zeros((L,), jnp.int32)

    copies = [
        pltpu.async_copy(
            x_hbm.at[base + i // HALVES, pl.ds((i % HALVES) * N2, N2)],
            buf.at[i],
            sems[i],
        )
        for i in range(SEG)
    ]

    for r in range(RPW):
        inf = jnp.full((L,), jnp.inf, jnp.float32)
        zero = jnp.zeros((L,), jnp.int32)
        carry = (inf,) * U + (zero,) * U

        for h in range(HALVES):
            seg = r * HALVES + h
            copies[seg].wait()

            def body(t, c, _seg=seg, _h=h):
                mvs = list(c[:U])
                tvs = list(c[U:])
                tvec = jnp.full((L,), t + _h * T2, jnp.int32)
                for j in range(U):
                    chunk = buf[_seg, pl.ds((t * U + j) * L, L)]
                    pred = chunk < mvs[j]
                    mvs[j] = jnp.where(pred, chunk, mvs[j])
                    tvs[j] = jnp.where(pred, tvec, tvs[j])
                return tuple(mvs) + tuple(tvs)

            carry = plsc.parallel_loop(0, T2, carry=carry, unroll=2)(body)

        mvs = list(carry[:U])
        tvs = list(carry[U:])
        cols = [(tvs[j] * U + j) * L + iota for j in range(U)]

        step = 1
        while step < U:
            for j in range(0, U, 2 * step):
                a, b = j, j + step
                take_b = (mvs[b] < mvs[a]) | (
                    (mvs[b] == mvs[a]) & (cols[b] < cols[a])
                )
                mvs[a] = jnp.where(take_b, mvs[b], mvs[a])
                cols[a] = jnp.where(take_b, cols[b], cols[a])
            step *= 2

        m = jnp.min(mvs[0])
        cand = jnp.where(mvs[0] == m, cols[0], jnp.int32(2**31 - 1))
        res = jnp.min(cand)
        sel = iota == r if r < RPW - 1 else iota >= r
        resvec = jnp.where(sel, jnp.full((L,), res, jnp.int32), resvec)

    res_v[...] = resvec
    idxs = base + jnp.minimum(iota, RPW - 1)
    pltpu.async_copy(res_v, out_hbm.at[idxs], sems[SEG]).wait()


def kernel(x):
    return _argmin_sc(x)


# speedup vs baseline: 3.7180x; 3.7180x over previous
import functools

import jax
import jax.numpy as jnp
from jax import lax
from jax.experimental import pallas as pl
from jax.experimental.pallas import tpu as pltpu
from jax.experimental.pallas import tpu_sc as plsc

R = 128
N = 8192
L = 16
NC = 1
NS = 16
NW = NC * NS
RPW = R // NW
HALVES = 2
SEG = RPW * HALVES
N2 = N // HALVES
U = 8
T2 = N2 // (L * U)

_mesh = plsc.VectorSubcoreMesh(
    core_axis_name="c", subcore_axis_name="s", num_cores=1
)


@functools.partial(
    pl.kernel,
    out_type=jax.ShapeDtypeStruct((R, L), jnp.int32),
    mesh=_mesh,
    compiler_params=pltpu.CompilerParams(
        needs_layout_passes=False,
        disable_bounds_checks=True,
        disable_semaphore_checks=True,
    ),
    scratch_types=[
        pltpu.VMEM((SEG, N2), jnp.float32),
        pltpu.VMEM((RPW, L), jnp.int32),
    ]
    + [pltpu.SemaphoreType.DMA] * SEG,
)
def _argmin_sc(x_hbm, out_hbm, buf, res_v, *sems):
    sid = lax.axis_index("s")
    base = sid * RPW
    iota = lax.iota(jnp.int32, L)

    copies = [
        pltpu.async_copy(
            x_hbm.at[base + i // HALVES, pl.ds((i % HALVES) * N2, N2)],
            buf.at[i],
            sems[i],
        )
        for i in range(SEG)
    ]

    for r in range(RPW):
        inf = jnp.full((L,), jnp.inf, jnp.float32)
        zero = jnp.zeros((L,), jnp.int32)
        carry = (inf,) * U + (zero,) * U

        for h in range(HALVES):
            seg = r * HALVES + h
            copies[seg].wait()

            def body(t, c, _seg=seg, _h=h):
                mvs = list(c[:U])
                tvs = list(c[U:])
                tvec = jnp.full((L,), t + _h * T2, jnp.int32)
                for j in range(U):
                    chunk = buf[_seg, pl.ds((t * U + j) * L, L)]
                    pred = chunk < mvs[j]
                    mvs[j] = jnp.where(pred, chunk, mvs[j])
                    tvs[j] = jnp.where(pred, tvec, tvs[j])
                return tuple(mvs) + tuple(tvs)

            carry = plsc.parallel_loop(0, T2, carry=carry, unroll=2)(body)

        mvs = list(carry[:U])
        tvs = list(carry[U:])
        cols = [(tvs[j] * U + j) * L + iota for j in range(U)]

        step = 1
        while step < U:
            for j in range(0, U, 2 * step):
                a, b = j, j + step
                take_b = (mvs[b] < mvs[a]) | (
                    (mvs[b] == mvs[a]) & (cols[b] < cols[a])
                )
                mvs[a] = jnp.where(take_b, mvs[b], mvs[a])
                cols[a] = jnp.where(take_b, cols[b], cols[a])
            step *= 2

        m = jnp.min(mvs[0])
        cand = jnp.where(mvs[0] == m, cols[0], jnp.int32(2**31 - 1))
        res = jnp.min(cand)
        res_v[r] = jnp.full((L,), res, jnp.int32)

    off = pl.multiple_of(base, RPW)
    pltpu.sync_copy(res_v, out_hbm.at[pl.ds(off, RPW)])


def kernel(x):
    return _argmin_sc(x)[:, 0]


# speedup vs baseline: 4.0847x; 1.0986x over previous
import functools

import jax
import jax.numpy as jnp
from jax import lax
from jax.experimental import pallas as pl
from jax.experimental.pallas import tpu as pltpu
from jax.experimental.pallas import tpu_sc as plsc

R = 128
N = 8192
L = 16
NC = 2
NS = 16
NW = NC * NS
RPW = R // NW
HALVES = 2
SEG = RPW * HALVES
N2 = N // HALVES
U = 8
T2 = N2 // (L * U)

_mesh = plsc.VectorSubcoreMesh(core_axis_name="c", subcore_axis_name="s")


@functools.partial(
    pl.kernel,
    out_type=jax.ShapeDtypeStruct((R, L), jnp.int32),
    mesh=_mesh,
    compiler_params=pltpu.CompilerParams(
        needs_layout_passes=False,
        disable_bounds_checks=True,
        disable_semaphore_checks=True,
    ),
    scratch_types=[
        pltpu.VMEM((SEG, N2), jnp.float32),
        pltpu.VMEM((RPW, L), jnp.int32),
    ]
    + [pltpu.SemaphoreType.DMA] * SEG,
)
def _argmin_sc(x_hbm, out_hbm, buf, res_v, *sems):
    cid = lax.axis_index("c")
    sid = lax.axis_index("s")
    base = (cid * NS + sid) * RPW
    iota = lax.iota(jnp.int32, L)

    copies = [
        pltpu.async_copy(
            x_hbm.at[base + i // HALVES, pl.ds((i % HALVES) * N2, N2)],
            buf.at[i],
            sems[i],
        )
        for i in range(SEG)
    ]

    for r in range(RPW):
        inf = jnp.full((L,), jnp.inf, jnp.float32)
        zero = jnp.zeros((L,), jnp.int32)
        carry = (inf,) * U + (zero,) * U

        for h in range(HALVES):
            seg = r * HALVES + h
            copies[seg].wait()

            def body(t, c, _seg=seg, _h=h):
                mvs = list(c[:U])
                tvs = list(c[U:])
                tvec = jnp.full((L,), t + _h * T2, jnp.int32)
                for j in range(U):
                    chunk = buf[_seg, pl.ds((t * U + j) * L, L)]
                    pred = chunk < mvs[j]
                    mvs[j] = jnp.where(pred, chunk, mvs[j])
                    tvs[j] = jnp.where(pred, tvec, tvs[j])
                return tuple(mvs) + tuple(tvs)

            carry = plsc.parallel_loop(0, T2, carry=carry, unroll=2)(body)

        mvs = list(carry[:U])
        tvs = list(carry[U:])
        cols = [(tvs[j] * U + j) * L + iota for j in range(U)]

        step = 1
        while step < U:
            for j in range(0, U, 2 * step):
                a, b = j, j + step
                take_b = (mvs[b] < mvs[a]) | (
                    (mvs[b] == mvs[a]) & (cols[b] < cols[a])
                )
                mvs[a] = jnp.where(take_b, mvs[b], mvs[a])
                cols[a] = jnp.where(take_b, cols[b], cols[a])
            step *= 2

        m = jnp.min(mvs[0])
        cand = jnp.where(mvs[0] == m, cols[0], jnp.int32(2**31 - 1))
        res = jnp.min(cand)
        res_v[r] = jnp.full((L,), res, jnp.int32)

    off = pl.multiple_of(base, RPW)
    pltpu.sync_copy(res_v, out_hbm.at[pl.ds(off, RPW)])


def kernel(x):
    return _argmin_sc(x)[:, 0]
